# baseline (speedup 1.0000x reference)
import functools

import jax
import jax.numpy as jnp
from jax import lax
from jax.experimental import pallas as pl
from jax.experimental.pallas import tpu as pltpu
from jax.experimental.pallas import tpu_sc as plsc

N = 10000
E = 320000
D = 128

NC = 2
NS = 16
NW = NC * NS
EPW = E // NW
K = 80
CB = 5
SB = 25
NP = 10240
RPT = NP // NS
NB = 4
KH = K // 2


def _sc_segment_sum(x, ei, zrows):
    mesh = plsc.VectorSubcoreMesh(core_axis_name="c", subcore_axis_name="s")

    @functools.partial(
        pl.kernel,
        out_type=jax.ShapeDtypeStruct((NC, NP, D), jnp.float32),
        mesh=mesh,
        scratch_types=[
            pltpu.VMEM((2, CB, K), jnp.int32),
            pltpu.VMEM((2, CB, K), jnp.int32),
            pltpu.VMEM((NB, K, D), jnp.float32),
            pltpu.VMEM_SHARED((NP, D), jnp.float32),
            pltpu.SemaphoreType.DMA,
            pltpu.SemaphoreType.DMA,
            pltpu.SemaphoreType.DMA,
        ],
    )
    def agg_kernel(x_hbm, ei_hbm, z_hbm, out_hbm, srcv, dstv, buf, agg_sh,
                   gsem, ssem, isem):
        cid = lax.axis_index("c")
        sid = lax.axis_index("s")
        wid = cid * NS + sid
        base = sid * RPT

        pltpu.sync_copy(ei_hbm.at[0, wid, 0], srcv.at[0])
        pltpu.sync_copy(ei_hbm.at[1, wid, 0], dstv.at[0])
        pltpu.async_copy(ei_hbm.at[0, wid, 1], srcv.at[1], isem)
        pltpu.async_copy(ei_hbm.at[1, wid, 1], dstv.at[1], isem)
        for b in range(NB - 1):
            pltpu.async_copy(x_hbm.at[srcv.at[0, b, pl.ds(0, KH)]],
                             buf.at[b, pl.ds(0, KH)], gsem)
            pltpu.async_copy(x_hbm.at[srcv.at[0, b, pl.ds(KH, KH)]],
                             buf.at[b, pl.ds(KH, KH)], gsem)
        pltpu.sync_copy(z_hbm, agg_sh.at[pl.ds(base, RPT)])
        plsc.subcore_barrier()

        for s in range(SB):
            p = s % 2
            q = (s + 1) % 2
            toff = s * CB

            def step_a(j, c, p=p, toff=toff):
                slot = lax.rem(toff + j, NB)
                pltpu.make_async_copy(x_hbm.at[srcv.at[p, j]],
                                      buf.at[slot], gsem).wait()
                ji = j + NB - 1
                nslot = lax.rem(toff + ji, NB)
                pltpu.async_copy(x_hbm.at[srcv.at[p, ji, pl.ds(0, KH)]],
                                 buf.at[nslot, pl.ds(0, KH)], gsem)
                pltpu.async_copy(x_hbm.at[srcv.at[p, ji, pl.ds(KH, KH)]],
                                 buf.at[nslot, pl.ds(KH, KH)], gsem)
                pltpu.sync_copy(buf.at[slot], agg_sh.at[dstv.at[p, j]],
                                add=True)
                return c

            lax.fori_loop(0, CB - NB + 1, step_a, 0)

            if s < SB - 1:
                pltpu.make_async_copy(ei_hbm.at[0, wid, s + 1],
                                      srcv.at[q], isem).wait()
                pltpu.make_async_copy(ei_hbm.at[1, wid, s + 1],
                                      dstv.at[q], isem).wait()

            def step_b(j, c, p=p, q=q, toff=toff, last=(s == SB - 1)):
                slot = lax.rem(toff + j, NB)
                pltpu.make_async_copy(x_hbm.at[srcv.at[p, j]],
                                      buf.at[slot], gsem).wait()
                if not last:
                    ji = j + NB - 1 - CB
                    nslot = lax.rem(toff + j + NB - 1, NB)
                    pltpu.async_copy(
                        x_hbm.at[srcv.at[q, ji, pl.ds(0, KH)]],
                        buf.at[nslot, pl.ds(0, KH)], gsem)
                    pltpu.async_copy(
                        x_hbm.at[srcv.at[q, ji, pl.ds(KH, KH)]],
                        buf.at[nslot, pl.ds(KH, KH)], gsem)
                pltpu.sync_copy(buf.at[slot], agg_sh.at[dstv.at[p, j]],
                                add=True)
                return c

            lax.fori_loop(CB - NB + 1, CB, step_b, 0)

            if s < SB - 2:
                pltpu.async_copy(ei_hbm.at[0, wid, s + 2], srcv.at[p],
                                 isem)
                pltpu.async_copy(ei_hbm.at[1, wid, s + 2], dstv.at[p],
                                 isem)

        plsc.subcore_barrier()
        pltpu.sync_copy(agg_sh.at[pl.ds(base, RPT)],
                        out_hbm.at[cid, pl.ds(base, RPT)])

    return agg_kernel(x, ei, zrows)


MB = 512


def _mlp_body(x_ref, p_ref, w1_ref, b1_ref, w2_ref, b2_ref, o_ref):
    h = x_ref[...] + p_ref[0] + p_ref[1]
    h = jnp.maximum(
        jnp.dot(h, w1_ref[...], preferred_element_type=jnp.float32)
        + b1_ref[...], 0.0)
    o_ref[...] = jnp.tanh(
        jnp.dot(h, w2_ref[...], preferred_element_type=jnp.float32)
        + b2_ref[...])


def _mlp(x, part, W1, b1, W2, b2):
    return pl.pallas_call(
        _mlp_body,
        grid=(pl.cdiv(N, MB),),
        in_specs=[
            pl.BlockSpec((MB, D), lambda i: (i, 0)),
            pl.BlockSpec((NC, MB, D), lambda i: (0, i, 0)),
            pl.BlockSpec((D, D), lambda i: (0, 0)),
            pl.BlockSpec((1, D), lambda i: (0, 0)),
            pl.BlockSpec((D, D), lambda i: (0, 0)),
            pl.BlockSpec((1, D), lambda i: (0, 0)),
        ],
        out_specs=pl.BlockSpec((MB, D), lambda i: (i, 0)),
        out_shape=jax.ShapeDtypeStruct((N, D), jnp.float32),
    )(x, part, W1, b1, W2, b2)


def kernel(x, edge_index, W1, b1, W2, b2):
    ei = edge_index.reshape(2, NW, SB, CB, K)
    zrows = jnp.zeros((RPT, D), jnp.float32)
    part = _sc_segment_sum(x, ei, zrows)
    return _mlp(x, part, W1, b1.reshape(1, D), W2, b2.reshape(1, D))


# speedup vs baseline: 1.0663x; 1.0663x over previous
import functools

import jax
import jax.numpy as jnp
from jax import lax
from jax.experimental import pallas as pl
from jax.experimental.pallas import tpu as pltpu
from jax.experimental.pallas import tpu_sc as plsc

N = 10000
E = 320000
D = 128

NC = 2
NS = 16
NW = NC * NS
EPW = E // NW
K = 80
CB = 5
SB = 25
NP = 10240
RPT = NP // NS
NB = 4


def _sc_segment_sum(x, ei, zrows):
    mesh = plsc.VectorSubcoreMesh(core_axis_name="c", subcore_axis_name="s")

    @functools.partial(
        pl.kernel,
        out_type=jax.ShapeDtypeStruct((NC, NP, D), jnp.float32),
        mesh=mesh,
        scratch_types=[
            pltpu.VMEM((2, CB, K), jnp.int32),
            pltpu.VMEM((2, CB, K), jnp.int32),
            pltpu.VMEM((NB, K, D), jnp.float32),
            pltpu.VMEM_SHARED((NP, D), jnp.float32),
            pltpu.SemaphoreType.DMA,
            pltpu.SemaphoreType.DMA,
            pltpu.SemaphoreType.DMA,
        ],
    )
    def agg_kernel(x_hbm, ei_hbm, z_hbm, out_hbm, srcv, dstv, buf, agg_sh,
                   gsem, ssem, isem):
        cid = lax.axis_index("c")
        sid = lax.axis_index("s")
        wid = cid * NS + sid
        base = sid * RPT

        pltpu.sync_copy(ei_hbm.at[0, wid, 0], srcv.at[0])
        pltpu.sync_copy(ei_hbm.at[1, wid, 0], dstv.at[0])
        pltpu.async_copy(ei_hbm.at[0, wid, 1], srcv.at[1], isem)
        pltpu.async_copy(ei_hbm.at[1, wid, 1], dstv.at[1], isem)
        for b in range(NB - 1):
            pltpu.async_copy(x_hbm.at[srcv.at[0, b]], buf.at[b], gsem)
        pltpu.sync_copy(z_hbm, agg_sh.at[pl.ds(base, RPT)])
        plsc.subcore_barrier()

        for s in range(SB):
            p = s % 2
            q = (s + 1) % 2
            toff = s * CB

            def step_a(j, c, p=p, toff=toff):
                slot = lax.rem(toff + j, NB)
                pltpu.make_async_copy(x_hbm.at[srcv.at[p, j]],
                                      buf.at[slot], gsem).wait()
                pltpu.async_copy(x_hbm.at[srcv.at[p, j + NB - 1]],
                                 buf.at[lax.rem(toff + j + NB - 1, NB)],
                                 gsem)
                pltpu.sync_copy(buf.at[slot], agg_sh.at[dstv.at[p, j]],
                                add=True)
                return c

            lax.fori_loop(0, CB - NB + 1, step_a, 0)

            if s < SB - 1:
                pltpu.make_async_copy(ei_hbm.at[0, wid, s + 1],
                                      srcv.at[q], isem).wait()
                pltpu.make_async_copy(ei_hbm.at[1, wid, s + 1],
                                      dstv.at[q], isem).wait()

            def step_b(j, c, p=p, q=q, toff=toff, last=(s == SB - 1)):
                slot = lax.rem(toff + j, NB)
                pltpu.make_async_copy(x_hbm.at[srcv.at[p, j]],
                                      buf.at[slot], gsem).wait()
                if not last:
                    pltpu.async_copy(
                        x_hbm.at[srcv.at[q, j + NB - 1 - CB]],
                        buf.at[lax.rem(toff + j + NB - 1, NB)], gsem)
                pltpu.sync_copy(buf.at[slot], agg_sh.at[dstv.at[p, j]],
                                add=True)
                return c

            lax.fori_loop(CB - NB + 1, CB, step_b, 0)

            if s < SB - 2:
                pltpu.async_copy(ei_hbm.at[0, wid, s + 2], srcv.at[p],
                                 isem)
                pltpu.async_copy(ei_hbm.at[1, wid, s + 2], dstv.at[p],
                                 isem)

        plsc.subcore_barrier()
        pltpu.sync_copy(agg_sh.at[pl.ds(base, RPT)],
                        out_hbm.at[cid, pl.ds(base, RPT)])

    return agg_kernel(x, ei, zrows)


MB = 2000


def _mlp_body(x_ref, p_ref, w1_ref, b1_ref, w2_ref, b2_ref, o_ref):
    h = x_ref[...] + p_ref[0] + p_ref[1]
    h = jnp.maximum(
        jnp.dot(h, w1_ref[...], preferred_element_type=jnp.float32)
        + b1_ref[...], 0.0)
    o_ref[...] = jnp.tanh(
        jnp.dot(h, w2_ref[...], preferred_element_type=jnp.float32)
        + b2_ref[...])


def _mlp(x, part, W1, b1, W2, b2):
    return pl.pallas_call(
        _mlp_body,
        grid=(pl.cdiv(N, MB),),
        in_specs=[
            pl.BlockSpec((MB, D), lambda i: (i, 0)),
            pl.BlockSpec((NC, MB, D), lambda i: (0, i, 0)),
            pl.BlockSpec((D, D), lambda i: (0, 0)),
            pl.BlockSpec((1, D), lambda i: (0, 0)),
            pl.BlockSpec((D, D), lambda i: (0, 0)),
            pl.BlockSpec((1, D), lambda i: (0, 0)),
        ],
        out_specs=pl.BlockSpec((MB, D), lambda i: (i, 0)),
        out_shape=jax.ShapeDtypeStruct((N, D), jnp.float32),
    )(x, part, W1, b1, W2, b2)


def kernel(x, edge_index, W1, b1, W2, b2):
    ei = edge_index.reshape(2, NW, SB, CB, K)
    zrows = jnp.zeros((RPT, D), jnp.float32)
    part = _sc_segment_sum(x, ei, zrows)
    return _mlp(x, part, W1, b1.reshape(1, D), W2, b2.reshape(1, D))


# speedup vs baseline: 1.0786x; 1.0116x over previous
import functools

import jax
import jax.numpy as jnp
from jax import lax
from jax.experimental import pallas as pl
from jax.experimental.pallas import tpu as pltpu
from jax.experimental.pallas import tpu_sc as plsc

N = 10000
E = 320000
D = 128

NC = 2
NS = 16
NW = NC * NS
EPW = E // NW
K = 80
CB = 5
SB = 25
NP = 10240
RPT = NP // NS
NB = 4


def _sc_segment_sum(x, ei, zrows):
    mesh = plsc.VectorSubcoreMesh(core_axis_name="c", subcore_axis_name="s")

    @functools.partial(
        pl.kernel,
        out_type=jax.ShapeDtypeStruct((NC, NP, D), jnp.float32),
        mesh=mesh,
        scratch_types=[
            pltpu.VMEM((2, CB, K), jnp.int32),
            pltpu.VMEM((2, CB, K), jnp.int32),
            pltpu.VMEM((NB, K, D), jnp.float32),
            pltpu.VMEM_SHARED((NP, D), jnp.float32),
            pltpu.SemaphoreType.DMA,
            pltpu.SemaphoreType.DMA,
            pltpu.SemaphoreType.DMA,
        ],
    )
    def agg_kernel(x_hbm, ei_hbm, z_hbm, out_hbm, srcv, dstv, buf, agg_sh,
                   gsem, ssem, isem):
        cid = lax.axis_index("c")
        sid = lax.axis_index("s")
        wid = cid * NS + sid
        base = sid * RPT

        pltpu.sync_copy(ei_hbm.at[0, wid, 0], srcv.at[0])
        pltpu.sync_copy(ei_hbm.at[1, wid, 0], dstv.at[0])
        pltpu.async_copy(ei_hbm.at[0, wid, 1], srcv.at[1], isem)
        pltpu.async_copy(ei_hbm.at[1, wid, 1], dstv.at[1], isem)
        for b in range(NB - 1):
            pltpu.async_copy(x_hbm.at[srcv.at[0, b]], buf.at[b], gsem)
        pltpu.sync_copy(z_hbm, agg_sh.at[pl.ds(base, RPT)])
        plsc.subcore_barrier()

        for s in range(SB):
            p = s % 2
            q = (s + 1) % 2
            toff = s * CB

            def step_a(j, c, p=p, toff=toff):
                slot = lax.rem(toff + j, NB)
                pltpu.make_async_copy(x_hbm.at[srcv.at[p, j]],
                                      buf.at[slot], gsem).wait()
                pltpu.async_copy(x_hbm.at[srcv.at[p, j + NB - 1]],
                                 buf.at[lax.rem(toff + j + NB - 1, NB)],
                                 gsem)
                pltpu.sync_copy(buf.at[slot], agg_sh.at[dstv.at[p, j]],
                                add=True)
                return c

            lax.fori_loop(0, CB - NB + 1, step_a, 0)

            if s < SB - 1:
                pltpu.make_async_copy(ei_hbm.at[0, wid, s + 1],
                                      srcv.at[q], isem).wait()
                pltpu.make_async_copy(ei_hbm.at[1, wid, s + 1],
                                      dstv.at[q], isem).wait()

            def step_b(j, c, p=p, q=q, toff=toff, last=(s == SB - 1)):
                slot = lax.rem(toff + j, NB)
                pltpu.make_async_copy(x_hbm.at[srcv.at[p, j]],
                                      buf.at[slot], gsem).wait()
                if not last:
                    pltpu.async_copy(
                        x_hbm.at[srcv.at[q, j + NB - 1 - CB]],
                        buf.at[lax.rem(toff + j + NB - 1, NB)], gsem)
                pltpu.sync_copy(buf.at[slot], agg_sh.at[dstv.at[p, j]],
                                add=True)
                return c

            lax.fori_loop(CB - NB + 1, CB, step_b, 0)

            if s < SB - 2:
                pltpu.async_copy(ei_hbm.at[0, wid, s + 2], srcv.at[p],
                                 isem)
                pltpu.async_copy(ei_hbm.at[1, wid, s + 2], dstv.at[p],
                                 isem)

        plsc.subcore_barrier()
        pltpu.sync_copy(agg_sh.at[pl.ds(base, RPT)],
                        out_hbm.at[cid, pl.ds(base, RPT)])

    return agg_kernel(x, ei, zrows)


MB = 5000


def _mlp_body(x_ref, p_ref, w1_ref, b1_ref, w2_ref, b2_ref, o_ref):
    h = x_ref[...] + p_ref[0] + p_ref[1]
    h = jnp.maximum(
        jnp.dot(h, w1_ref[...], preferred_element_type=jnp.float32)
        + b1_ref[...], 0.0)
    o_ref[...] = jnp.tanh(
        jnp.dot(h, w2_ref[...], preferred_element_type=jnp.float32)
        + b2_ref[...])


def _mlp(x, part, W1, b1, W2, b2):
    return pl.pallas_call(
        _mlp_body,
        grid=(pl.cdiv(N, MB),),
        in_specs=[
            pl.BlockSpec((MB, D), lambda i: (i, 0)),
            pl.BlockSpec((NC, MB, D), lambda i: (0, i, 0)),
            pl.BlockSpec((D, D), lambda i: (0, 0)),
            pl.BlockSpec((1, D), lambda i: (0, 0)),
            pl.BlockSpec((D, D), lambda i: (0, 0)),
            pl.BlockSpec((1, D), lambda i: (0, 0)),
        ],
        out_specs=pl.BlockSpec((MB, D), lambda i: (i, 0)),
        out_shape=jax.ShapeDtypeStruct((N, D), jnp.float32),
    )(x, part, W1, b1, W2, b2)


def kernel(x, edge_index, W1, b1, W2, b2):
    ei = edge_index.reshape(2, NW, SB, CB, K)
    zrows = jnp.zeros((RPT, D), jnp.float32)
    part = _sc_segment_sum(x, ei, zrows)
    return _mlp(x, part, W1, b1.reshape(1, D), W2, b2.reshape(1, D))
